# baseline (speedup 1.0000x reference)
import jax
import jax.numpy as jnp
from jax.experimental import pallas as pl
from jax.experimental.pallas import tpu as pltpu

_T = 2048
_D = 1024
_E = 8
_NQ = 8
_QW = _D // _NQ


def _fused_kernel(x_ref, wa_ref, bias_ref, ws_ref, bs_ref, out_ref,
                  xb_ref, w_ref):
    e = pl.program_id(0)

    @pl.when(e == 0)
    def _prologue():
        x32 = x_ref[...]
        logits = jnp.dot(x32, wa_ref[...],
                         preferred_element_type=jnp.float32) + bias_ref[...]
        w_ref[...] = jax.nn.softmax(logits, axis=-1)
        xb_ref[...] = x32.astype(jnp.bfloat16)
        out_ref[...] = jnp.zeros((_T, _D), jnp.float32)

    w = w_ref[...]
    lane = jax.lax.broadcasted_iota(jnp.int32, w.shape, 1)
    we = jnp.sum(jnp.where(lane == e, w, 0.0), axis=1, keepdims=True)
    xb = xb_ref[...]
    for q in range(_NQ):
        qsl = pl.ds(q * _QW, _QW)
        wq = ws_ref[0, :, qsl].astype(jnp.bfloat16)
        h = jnp.dot(xb, wq, preferred_element_type=jnp.float32)
        c = we * jnp.tanh(h + bs_ref[0, :, qsl])
        out_ref[:, qsl] = out_ref[:, qsl] + c


def kernel(x, W_attn, b_attn, adaptive_bias, W_s, b_s):
    bias = (b_attn + adaptive_bias).reshape(1, _E)
    return pl.pallas_call(
        _fused_kernel,
        grid=(_E,),
        in_specs=[
            pl.BlockSpec((_T, _D), lambda e: (0, 0)),
            pl.BlockSpec((_D, _E), lambda e: (0, 0)),
            pl.BlockSpec((1, _E), lambda e: (0, 0)),
            pl.BlockSpec((1, _D, _D), lambda e: (e, 0, 0)),
            pl.BlockSpec((1, 1, _D), lambda e: (e, 0, 0)),
        ],
        out_specs=pl.BlockSpec((_T, _D), lambda e: (0, 0)),
        out_shape=jax.ShapeDtypeStruct((_T, _D), jnp.float32),
        scratch_shapes=[
            pltpu.VMEM((_T, _D), jnp.bfloat16),
            pltpu.VMEM((_T, _E), jnp.float32),
        ],
        compiler_params=pltpu.CompilerParams(
            dimension_semantics=("arbitrary",),
        ),
    )(x, W_attn, bias, W_s, b_s.reshape(_E, 1, _D))


# speedup vs baseline: 1.5895x; 1.5895x over previous
import jax
import jax.numpy as jnp
from jax.experimental import pallas as pl
from jax.experimental.pallas import tpu as pltpu

_T = 2048
_D = 1024
_E = 8
_NQ = 2
_QW = _D // _NQ


def _fused_kernel(x_ref, wa_ref, bias_ref, ws_ref, bs_ref, out_ref,
                  xb_ref, w_ref):
    e = pl.program_id(0)

    @pl.when(e == 0)
    def _prologue():
        x32 = x_ref[...]
        logits = jnp.dot(x32, wa_ref[...],
                         preferred_element_type=jnp.float32) + bias_ref[...]
        w_ref[...] = jax.nn.softmax(logits, axis=-1)
        xb_ref[...] = x32.astype(jnp.bfloat16)
        out_ref[...] = jnp.zeros((_T, _D), jnp.float32)

    w = w_ref[...]
    lane = jax.lax.broadcasted_iota(jnp.int32, w.shape, 1)
    we = jnp.sum(jnp.where(lane == e, w, 0.0), axis=1, keepdims=True)
    xb = xb_ref[...]
    for q in range(_NQ):
        qsl = pl.ds(q * _QW, _QW)
        wq = ws_ref[0, :, qsl].astype(jnp.bfloat16)
        h = jnp.dot(xb, wq, preferred_element_type=jnp.float32)
        c = we * jnp.tanh(h + bs_ref[0, :, qsl])
        out_ref[:, qsl] = out_ref[:, qsl] + c


def kernel(x, W_attn, b_attn, adaptive_bias, W_s, b_s):
    bias = (b_attn + adaptive_bias).reshape(1, _E)
    return pl.pallas_call(
        _fused_kernel,
        grid=(_E,),
        in_specs=[
            pl.BlockSpec((_T, _D), lambda e: (0, 0)),
            pl.BlockSpec((_D, _E), lambda e: (0, 0)),
            pl.BlockSpec((1, _E), lambda e: (0, 0)),
            pl.BlockSpec((1, _D, _D), lambda e: (e, 0, 0)),
            pl.BlockSpec((1, 1, _D), lambda e: (e, 0, 0)),
        ],
        out_specs=pl.BlockSpec((_T, _D), lambda e: (0, 0)),
        out_shape=jax.ShapeDtypeStruct((_T, _D), jnp.float32),
        scratch_shapes=[
            pltpu.VMEM((_T, _D), jnp.bfloat16),
            pltpu.VMEM((_T, _E), jnp.float32),
        ],
        compiler_params=pltpu.CompilerParams(
            dimension_semantics=("arbitrary",),
        ),
    )(x, W_attn, bias, W_s, b_s.reshape(_E, 1, _D))
